# Initial kernel scaffold; baseline (speedup 1.0000x reference)
import jax
import jax.numpy as jnp
from jax.experimental import pallas as pl


def kernel(x, edge_index, W1, b1, W2, b2):
    raise NotImplementedError("write your pallas kernel here")


# speedup vs baseline: 23.0752x; 23.0752x over previous
import functools

import jax
import jax.numpy as jnp
from jax import lax
from jax.experimental import pallas as pl
from jax.experimental.pallas import tpu as pltpu
from jax.experimental.pallas import tpu_sc as plsc

_N = 10000
_E = 320000
_DIM = 16

_NW = 32
_EPT_RAW = _E // _NW
_B = 128
_NB = -(-_EPT_RAW // _B)
_EPT = _NB * _B
_PAD = _EPT - _EPT_RAW
_RPT = 632
_ACC_ROWS = _RPT * 16

@functools.cache
def _sc_kernels():
    mesh = plsc.VectorSubcoreMesh(
        core_axis_name="c", subcore_axis_name="s", num_cores=2, num_subcores=16
    )

    @functools.partial(
        pl.kernel,
        out_type=jax.ShapeDtypeStruct((_NW * _ACC_ROWS,), jnp.float32),
        mesh=mesh,
        scratch_types=[
            pltpu.VMEM((_EPT,), jnp.int32),
            pltpu.VMEM((_ACC_ROWS,), jnp.float32),
        ],
        compiler_params=pltpu.CompilerParams(needs_layout_passes=False),
    )
    def sc_degree(dst_hbm, out_hbm, didx, deg):
        wid = lax.axis_index("c") * 16 + lax.axis_index("s")
        zeros = jnp.zeros((16,), jnp.float32)

        def zbody(i, _):
            deg[pl.ds(i * 16, 16)] = zeros
            return 0

        lax.fori_loop(0, _ACC_ROWS // 16, zbody, 0)
        pltpu.sync_copy(dst_hbm.at[pl.ds(wid * _EPT, _EPT)], didx)
        ones = jnp.ones((16,), jnp.float32)

        def body(i, _):
            idx = didx[pl.ds(i * 16, 16)]
            plsc.addupdate_scatter(deg, [idx], ones)
            return 0

        lax.fori_loop(0, _EPT // 16, body, 0)
        pltpu.sync_copy(deg, out_hbm.at[pl.ds(wid * _ACC_ROWS, _ACC_ROWS)])

    @functools.partial(
        pl.kernel,
        out_type=jax.ShapeDtypeStruct((2, _ACC_ROWS, _DIM), jnp.float32),
        mesh=mesh,
        scratch_types=[
            pltpu.VMEM((_B,), jnp.int32),
            pltpu.VMEM((_B,), jnp.int32),
            pltpu.VMEM((_B, _DIM), jnp.float32),
            pltpu.VMEM((_RPT, _DIM), jnp.float32),
            pltpu.VMEM_SHARED((_ACC_ROWS, _DIM), jnp.float32),
            pltpu.SemaphoreType.DMA,
        ],
        compiler_params=pltpu.CompilerParams(use_tc_tiling_on_sc=False),
    )
    def sc_agg(tab_hbm, src_hbm, dst_hbm, out_hbm, sidx, didx, rows, buf, acc, sem):
        c = lax.axis_index("c")
        s = lax.axis_index("s")
        wid = c * 16 + s
        zeros = jnp.zeros((16,), jnp.float32)

        def zbody(i, _):
            buf[i, :] = zeros
            return 0

        lax.fori_loop(0, _RPT, zbody, 0)
        pltpu.sync_copy(buf, acc.at[pl.ds(s * _RPT, _RPT)])
        plsc.subcore_barrier()

        ebase = wid * _EPT

        def body(g, _):
            base = ebase + g * _B
            pltpu.sync_copy(src_hbm.at[pl.ds(base, _B)], sidx)
            pltpu.sync_copy(dst_hbm.at[pl.ds(base, _B)], didx)
            pltpu.async_copy(tab_hbm.at[sidx], rows, sem).wait()
            pltpu.sync_copy(rows, acc.at[didx], add=True)
            return 0

        lax.fori_loop(0, _NB, body, 0)
        plsc.subcore_barrier()
        pltpu.sync_copy(acc.at[pl.ds(s * _RPT, _RPT)], buf)
        pltpu.sync_copy(buf, out_hbm.at[c, pl.ds(s * _RPT, _RPT)])

    return sc_degree, sc_agg


def _tc1_body(degp_ref, x_ref, w1_ref, dis_ref, hs_ref, hself_ref):
    deg = jnp.sum(degp_ref[:, :_N], axis=0) + 1.0
    dis = lax.rsqrt(deg)
    h = jnp.dot(x_ref[...], w1_ref[...], preferred_element_type=jnp.float32)
    d2 = dis[:, None]
    hs = h * d2
    dis_ref[...] = dis
    hs_ref[...] = hs
    hself_ref[...] = hs * d2


_tc1 = pl.pallas_call(
    _tc1_body,
    out_shape=(
        jax.ShapeDtypeStruct((_N,), jnp.float32),
        jax.ShapeDtypeStruct((_N, _DIM), jnp.float32),
        jax.ShapeDtypeStruct((_N, _DIM), jnp.float32),
    ),
)


def _tc2_body(acc_ref, dis_ref, hself_ref, b1_ref, w2_ref, gs_ref, gself_ref):
    dis = dis_ref[...][:, None]
    z = (acc_ref[0, :_N, :] + acc_ref[1, :_N, :]) * dis + hself_ref[...] + b1_ref[...][None, :]
    h2 = jnp.maximum(z, 0.0)
    g = jnp.dot(h2, w2_ref[...], preferred_element_type=jnp.float32)
    gs = g * dis
    gs_ref[...] = gs
    gself_ref[...] = gs * dis


_tc2 = pl.pallas_call(
    _tc2_body,
    out_shape=(
        jax.ShapeDtypeStruct((_N, _DIM), jnp.float32),
        jax.ShapeDtypeStruct((_N, _DIM), jnp.float32),
    ),
)


def _tc3_body(acc_ref, dis_ref, gself_ref, b2_ref, out_ref):
    dis = dis_ref[...][:, None]
    logits = (acc_ref[0, :_N, :] + acc_ref[1, :_N, :]) * dis + gself_ref[...] + b2_ref[...][None, :]
    m = jnp.max(logits, axis=1, keepdims=True)
    lse = jnp.log(jnp.sum(jnp.exp(logits - m), axis=1, keepdims=True)) + m
    out_ref[...] = logits - lse


_tc3 = pl.pallas_call(
    _tc3_body,
    out_shape=jax.ShapeDtypeStruct((_N, _DIM), jnp.float32),
)


def kernel(x, edge_index, W1, b1, W2, b2):
    ei = edge_index.astype(jnp.int32)
    src = ei[0].reshape(_NW, _EPT_RAW)
    dst = ei[1].reshape(_NW, _EPT_RAW)
    srcp = jnp.pad(src, ((0, 0), (0, _PAD))).reshape(-1)
    dstp = jnp.pad(dst, ((0, 0), (0, _PAD)), constant_values=_N).reshape(-1)

    sc_degree, sc_agg = _sc_kernels()
    degp = sc_degree(dstp).reshape(_NW, _ACC_ROWS)
    dis, hs, hself = _tc1(degp, x, W1)
    acc1 = sc_agg(hs, srcp, dstp)
    gs, gself = _tc2(acc1, dis, hself, b1, W2)
    acc2 = sc_agg(gs, srcp, dstp)
    return _tc3(acc2, dis, gself, b2)
